# baseline (speedup 1.0000x reference)
import functools

import jax
import jax.numpy as jnp
from jax import lax
from jax.experimental import pallas as pl
from jax.experimental.pallas import tpu as pltpu
from jax.experimental.pallas import tpu_sc as plsc

NC = 2
NS = 16
NW = NC * NS
LANES = 16
CHUNK = 128


@functools.lru_cache(maxsize=None)
def _build(n_tokens: int, emb: int, start: int, part: int, zero_row: int):
    b_per_w = n_tokens // NW
    n_chunks = b_per_w // CHUNK
    mesh = plsc.VectorSubcoreMesh(core_axis_name="c", subcore_axis_name="s")

    @functools.partial(
        pl.kernel,
        mesh=mesh,
        out_type=jax.ShapeDtypeStruct((n_tokens, emb), jnp.float32),
        scratch_types=[
            pltpu.VMEM((b_per_w,), jnp.int32),
            pltpu.VMEM((CHUNK, emb), jnp.float32),
            pltpu.SemaphoreType.DMA,
        ],
    )
    def k(ids_hbm, table_hbm, out_hbm, idx_v, rows_v, sem):
        wid = lax.axis_index("s") * NC + lax.axis_index("c")
        base = wid * b_per_w
        pltpu.sync_copy(ids_hbm.at[pl.ds(base, b_per_w)], idx_v)

        def xform(i, carry):
            sl = pl.ds(i * LANES, LANES)
            v = idx_v[sl]
            ok = (v >= start) & (v < start + part)
            idx_v[sl] = jnp.where(ok, v - start, zero_row)
            return carry

        lax.fori_loop(0, b_per_w // LANES, xform, 0)

        def chunk(c, carry):
            off = c * CHUNK
            pltpu.async_copy(
                table_hbm.at[idx_v.at[pl.ds(off, CHUNK)]], rows_v, sem
            ).wait()
            pltpu.sync_copy(rows_v, out_hbm.at[pl.ds(base + off, CHUNK)])
            return carry

        lax.fori_loop(0, n_chunks, chunk, 0)

    return k


def kernel(input_ids, weight):
    b, s = input_ids.shape
    part, emb = weight.shape
    rank = 1
    start = rank * part
    ids = input_ids.reshape(-1).astype(jnp.int32)
    table = jnp.concatenate(
        [weight, jnp.zeros((8, emb), weight.dtype)], axis=0
    )
    out = _build(b * s, emb, start, part, part)(ids, table)
    return out.reshape(b, s, emb)
